# baseline (speedup 1.0000x reference)
import functools

import jax
import jax.numpy as jnp
from jax import lax
from jax.experimental import pallas as pl
from jax.experimental.pallas import tpu as pltpu
from jax.experimental.pallas import tpu_sc as plsc

_NC = 2
_NS = 16
_BM = 256


def _make_deg(Npad, E):
  EperW = E // (_NC * _NS)
  NB = (EperW + 15) // 16
  mesh = plsc.VectorSubcoreMesh(core_axis_name="c", subcore_axis_name="s")

  def body(dsth, out, dstv, hist):
    c = lax.axis_index("c")
    s = lax.axis_index("s")
    wid = s * _NC + c
    pltpu.sync_copy(dsth.at[pl.ds(wid * EperW, EperW)],
                    dstv.at[pl.ds(0, EperW)])
    zero = jnp.zeros((16,), jnp.float32)

    def zb(i, _):
      hist[pl.ds(i * 16, 16)] = zero
      return 0

    lax.fori_loop(0, Npad // 16, zb, 0)
    ones = jnp.ones((16,), jnp.float32)
    lanes = lax.iota(jnp.int32, 16)

    def sb(i, _):
      idx = dstv[pl.ds(i * 16, 16)]
      m = (i * 16 + lanes) < EperW
      idx = jnp.where(m, idx, 0)
      plsc.addupdate_scatter(hist, [idx], ones, mask=m)
      return 0

    lax.fori_loop(0, NB, sb, 0)
    pltpu.sync_copy(hist, out.at[wid])

  return pl.kernel(
      body,
      out_type=jax.ShapeDtypeStruct((_NC * _NS, Npad), jnp.float32),
      mesh=mesh,
      scratch_types=[
          pltpu.VMEM((NB * 16,), jnp.int32),
          pltpu.VMEM((Npad,), jnp.float32),
      ],
      compiler_params=pltpu.CompilerParams(needs_layout_passes=False),
  )


def _make_agg(Npad, E, W, ACCR, SUB, K):
  NT = _NC * _NS
  RT = Npad // NT
  NSEG = 16 if W <= 512 else 25
  EperS = E // NSEG
  CAP = EperS + 16
  REG = NSEG * CAP + CAP
  assert RT == ACCR * SUB and EperS % 16 == 0 and K % 16 == 0
  SENT = jnp.int32(RT << 14)
  mesh = plsc.VectorSubcoreMesh(core_axis_name="c", subcore_axis_name="s")

  def body(vals, srch, dsth, outf, creg, sbuf, dbuf, cbuf, sec, rsrc, rsrc2,
           gbuf, gbuf2, acc, sem, sem2):
    c = lax.axis_index("c")
    s = lax.axis_index("s")
    wid = s * _NC + c
    rt_lo = wid * RT
    lanes = lax.iota(jnp.int32, 16)

    def seg_body(g, off):
      pltpu.sync_copy(srch.at[pl.ds(g * EperS, EperS)], sbuf)
      pltpu.sync_copy(dsth.at[pl.ds(g * EperS, EperS)],
                      dbuf.at[pl.ds(0, EperS)])

      def cb(i, cnt):
        d = dbuf[pl.ds(i * 16, 16)] - rt_lo
        m = (d >= 0) & (d < RT)
        pk = sbuf[pl.ds(i * 16, 16)] | (d << 14)
        plsc.store_compressed(cbuf.at[pl.ds(cnt, 16)], pk, mask=m)
        return cnt + plsc.all_reduce_population_count(m)[0]

      cnt = lax.fori_loop(0, EperS // 16, cb, jnp.int32(0))
      cbuf[pl.ds(cnt, 16)] = jnp.full((16,), SENT, jnp.int32)
      pltpu.sync_copy(
          cbuf, creg.at[pl.ds(pl.multiple_of(wid * REG + off, 8), CAP)])
      return off + ((cnt + 7) & jnp.int32(-8))

    total = lax.fori_loop(0, NSEG, seg_body, jnp.int32(0))

    def sub_body(sub, _):
      base_d = sub * ACCR
      dummy = jnp.zeros((16,), jnp.int32) + ((base_d + ACCR) << 14)

      def zb(i, _):
        acc[pl.ds(i * 16, 16)] = jnp.zeros((16,), jnp.float32)
        return 0

      lax.fori_loop(0, ((ACCR + 1) * W) // 16, zb, 0)
      nb1 = (total + CAP - 1) // CAP

      def rb_body(t, _):
        pltpu.sync_copy(
            creg.at[pl.ds(pl.multiple_of(wid * REG + t * CAP, 8), CAP)], dbuf)
        hi = jnp.minimum(total - t * CAP, CAP)

        def sc(i, sc_cnt):
          pk = dbuf[pl.ds(i * 16, 16)]
          dl = pk >> 14
          m = ((i * 16 + lanes) < hi) & (dl >= base_d) & (dl < base_d + ACCR)
          plsc.store_compressed(sec.at[pl.ds(sc_cnt, 16)], pk, mask=m)
          return sc_cnt + plsc.all_reduce_population_count(m)[0]

        scnt = lax.fori_loop(0, (hi + 15) // 16, sc, jnp.int32(0))
        for j in range(K // 16):
          sec[pl.ds(scnt + j * 16, 16)] = dummy
        nb2 = (scnt + K - 1) // K

        def fire(i, rs, gb, sm):
          for j in range(K // 16):
            pv = sec[pl.ds(i * K + j * 16, 16)]
            rs[pl.ds(j * 16, 16)] = pv & 0x3FFF
          pltpu.async_copy(vals.at[rs], gb, sm)

        def wait_g(rs, gb, sm):
          pltpu.make_async_copy(vals.at[rs], gb, sm).wait()

        def accum(i, gb):
          bvecs = []
          for j in range(K // 16):
            pv = sec[pl.ds(i * K + j * 16, 16)]
            dv = ((pv >> 14) - base_d) * W
            for r in range(16):
              lane_r = jnp.full((16,), r, jnp.int32)
              bvecs.append(
                  lanes + dv.at[lane_r].get(mode="promise_in_bounds"))

          def jb(j, _):
            col = j * 16
            for r in range(K):
              plsc.addupdate_scatter(acc, [bvecs[r] + col],
                                     gb[r, pl.ds(col, 16)])
            return 0

          lax.fori_loop(0, W // 16, jb, 0)

        @pl.when(nb2 > 0)
        def _():
          fire(0, rsrc, gbuf, sem)

        def drain2(b2, _):
          b = 2 * b2
          wait_g(rsrc, gbuf, sem)

          @pl.when(b + 1 < nb2)
          def _():
            fire(b + 1, rsrc2, gbuf2, sem2)

          accum(b, gbuf)

          @pl.when(b + 1 < nb2)
          def _():
            wait_g(rsrc2, gbuf2, sem2)

            @pl.when(b + 2 < nb2)
            def _():
              fire(b + 2, rsrc, gbuf, sem)

            accum(b + 1, gbuf2)

          return 0

        lax.fori_loop(0, (nb2 + 1) // 2, drain2, 0)
        return 0

      lax.fori_loop(0, nb1, rb_body, 0)
      pltpu.sync_copy(
          acc.at[pl.ds(0, ACCR * W)],
          outf.at[pl.ds(pl.multiple_of((rt_lo + base_d) * W, 8), ACCR * W)])
      return 0

    lax.fori_loop(0, SUB, sub_body, 0)

  return pl.kernel(
      body,
      out_type=(
          jax.ShapeDtypeStruct((Npad * W,), jnp.float32),
          jax.ShapeDtypeStruct((NT * REG,), jnp.int32),
      ),
      mesh=mesh,
      scratch_types=[
          pltpu.VMEM((EperS,), jnp.int32),
          pltpu.VMEM((CAP,), jnp.int32),
          pltpu.VMEM((CAP,), jnp.int32),
          pltpu.VMEM((CAP + K,), jnp.int32),
          pltpu.VMEM((K,), jnp.int32),
          pltpu.VMEM((K,), jnp.int32),
          pltpu.VMEM((K, W), jnp.float32),
          pltpu.VMEM((K, W), jnp.float32),
          pltpu.VMEM(((ACCR + 1) * W,), jnp.float32),
          pltpu.SemaphoreType.DMA,
          pltpu.SemaphoreType.DMA,
      ],
      compiler_params=pltpu.CompilerParams(needs_layout_passes=False),
  )


def _prep(deg3, x_pad):
  Npad, IN = x_pad.shape
  NBLK = Npad // _BM

  def body(deg_ref, x_ref, dinv_ref, xs_ref):
    deg = jnp.sum(deg_ref[...], axis=0) + 1.0
    dinv = lax.rsqrt(deg)
    dinv_ref[...] = dinv
    xs_ref[...] = x_ref[...] * dinv

  return pl.pallas_call(
      body,
      grid=(NBLK,),
      in_specs=[
          pl.BlockSpec((_NC * _NS, _BM, 1), lambda i: (0, i, 0)),
          pl.BlockSpec((_BM, IN), lambda i: (i, 0)),
      ],
      out_specs=[
          pl.BlockSpec((_BM, 1), lambda i: (i, 0)),
          pl.BlockSpec((_BM, IN), lambda i: (i, 0)),
      ],
      out_shape=[
          jax.ShapeDtypeStruct((Npad, 1), jnp.float32),
          jax.ShapeDtypeStruct((Npad, IN), jnp.float32),
      ],
  )(deg3, x_pad)


def _mm1(scat1, xs, dinv, W1, b1):
  Npad, IN = xs.shape
  H = W1.shape[1]
  NBLK = Npad // _BM

  def body(s_ref, x_ref, d_ref, w_ref, b_ref, o_ref):
    dv = d_ref[...]
    a = (s_ref[...] + x_ref[...]) * dv
    z = jnp.dot(a, w_ref[...], preferred_element_type=jnp.float32) + b_ref[...]
    o_ref[...] = jnp.maximum(z, 0.0) * dv

  return pl.pallas_call(
      body,
      grid=(NBLK,),
      in_specs=[
          pl.BlockSpec((_BM, IN), lambda i: (i, 0)),
          pl.BlockSpec((_BM, IN), lambda i: (i, 0)),
          pl.BlockSpec((_BM, 1), lambda i: (i, 0)),
          pl.BlockSpec((IN, H), lambda i: (0, 0)),
          pl.BlockSpec((1, H), lambda i: (0, 0)),
      ],
      out_specs=pl.BlockSpec((_BM, H), lambda i: (i, 0)),
      out_shape=jax.ShapeDtypeStruct((Npad, H), jnp.float32),
  )(scat1, xs, dinv, W1, b1.reshape(1, -1))


def _mm2(scat2, h1s, dinv, W2, b2, Wc):
  Npad, H = h1s.shape
  OUT = Wc.shape[1]
  NBLK = Npad // _BM

  def body(s_ref, h_ref, d_ref, w_ref, b_ref, wc_ref, o_ref):
    dv = d_ref[...]
    a = (s_ref[...] + h_ref[...]) * dv
    z = jnp.dot(a, w_ref[...], preferred_element_type=jnp.float32) + b_ref[...]
    h2 = jnp.maximum(z, 0.0)
    o_ref[...] = jnp.dot(h2, wc_ref[...], preferred_element_type=jnp.float32) * dv

  return pl.pallas_call(
      body,
      grid=(NBLK,),
      in_specs=[
          pl.BlockSpec((_BM, H), lambda i: (i, 0)),
          pl.BlockSpec((_BM, H), lambda i: (i, 0)),
          pl.BlockSpec((_BM, 1), lambda i: (i, 0)),
          pl.BlockSpec((H, H), lambda i: (0, 0)),
          pl.BlockSpec((1, H), lambda i: (0, 0)),
          pl.BlockSpec((H, OUT), lambda i: (0, 0)),
      ],
      out_specs=pl.BlockSpec((_BM, OUT), lambda i: (i, 0)),
      out_shape=jax.ShapeDtypeStruct((Npad, OUT), jnp.float32),
  )(scat2, h1s, dinv, W2, b2.reshape(1, -1), Wc)


def _wc(W3, Wl):
  def body(w3_ref, wl_ref, o_ref):
    o_ref[...] = jnp.dot(w3_ref[...], wl_ref[...],
                         preferred_element_type=jnp.float32)

  return pl.pallas_call(
      body,
      out_shape=jax.ShapeDtypeStruct((W3.shape[0], Wl.shape[1]), jnp.float32),
  )(W3, Wl)


def _pool(scat3, ms, dinv, batch3, b3, Wl, bl, G):
  Npad, OUT = scat3.shape
  H = Wl.shape[0]
  NBLK = Npad // _BM

  def body(s_ref, m_ref, d_ref, b_ref, b3_ref, wl_ref, bl_ref, o_ref,
           sums, cnts):
    i = pl.program_id(0)

    @pl.when(i == 0)
    def _():
      sums[...] = jnp.zeros_like(sums)
      cnts[...] = jnp.zeros_like(cnts)

    a3 = (s_ref[...] + m_ref[...]) * d_ref[...]
    b = b_ref[0]
    oh = (lax.broadcasted_iota(jnp.int32, (G, _BM), 0) == b).astype(jnp.float32)
    sums[...] += jnp.dot(oh, a3, preferred_element_type=jnp.float32)
    cnts[...] += jnp.dot(oh, jnp.ones((_BM, OUT), jnp.float32),
                         preferred_element_type=jnp.float32)

    @pl.when(i == NBLK - 1)
    def _():
      c = cnts[...]
      bc = jnp.dot(b3_ref[...], wl_ref[...], preferred_element_type=jnp.float32)
      o_ref[...] = (sums[...] / jnp.maximum(c, 1.0)
                    + jnp.where(c > 0.0, bc, 0.0) + bl_ref[...])

  return pl.pallas_call(
      body,
      grid=(NBLK,),
      in_specs=[
          pl.BlockSpec((_BM, OUT), lambda i: (i, 0)),
          pl.BlockSpec((_BM, OUT), lambda i: (i, 0)),
          pl.BlockSpec((_BM, 1), lambda i: (i, 0)),
          pl.BlockSpec((1, 1, _BM), lambda i: (i, 0, 0)),
          pl.BlockSpec((1, H), lambda i: (0, 0)),
          pl.BlockSpec((H, OUT), lambda i: (0, 0)),
          pl.BlockSpec((1, OUT), lambda i: (0, 0)),
      ],
      out_specs=pl.BlockSpec((G, OUT), lambda i: (0, 0)),
      out_shape=jax.ShapeDtypeStruct((G, OUT), jnp.float32),
      scratch_shapes=[
          pltpu.VMEM((G, OUT), jnp.float32),
          pltpu.VMEM((G, OUT), jnp.float32),
      ],
  )(scat3, ms, dinv, batch3, b3.reshape(1, -1), Wl, bl.reshape(1, -1))


def kernel(x, edge_index, batch, W1, b1, W2, b2, W3, b3, Wl, bl):
  N, IN = x.shape
  E = edge_index.shape[1]
  H = W1.shape[1]
  OUT = Wl.shape[1]
  G = 64
  Npad = ((N + 2559) // 2560) * 2560

  src = edge_index[0]
  dst = edge_index[1]
  x_pad = jnp.pad(x.astype(jnp.float32), ((0, Npad - N), (0, 0)))
  batch_pad = jnp.pad(batch, (0, Npad - N), constant_values=G)
  batch3 = batch_pad.reshape(Npad // _BM, 1, _BM)

  deg = _make_deg(Npad, E)(dst)
  dinv, xs = _prep(deg.reshape(_NC * _NS, Npad, 1), x_pad)

  RT = Npad // (_NC * _NS)
  scat1, _ = _make_agg(Npad, E, IN, RT // 2, 2, 64)(xs, src, dst)
  h1s = _mm1(scat1.reshape(Npad, IN), xs, dinv, W1, b1)

  scat2, _ = _make_agg(Npad, E, H, RT // 10, 10, 32)(h1s, src, dst)
  Wc = _wc(W3, Wl)
  Wcp = jnp.pad(Wc, ((0, 0), (0, 128 - OUT)))
  ms128 = _mm2(scat2.reshape(Npad, H), h1s, dinv, W2, b2, Wcp)

  scat3, _ = _make_agg(Npad, E, 128, RT, 1, 64)(ms128, src, dst)
  return _pool(scat3.reshape(Npad, 128)[:, :OUT], ms128[:, :OUT], dinv,
               batch3, b3, Wl, bl, G)


# speedup vs baseline: 1.0136x; 1.0136x over previous
import functools

import jax
import jax.numpy as jnp
from jax import lax
from jax.experimental import pallas as pl
from jax.experimental.pallas import tpu as pltpu
from jax.experimental.pallas import tpu_sc as plsc

_NC = 2
_NS = 16
_BM = 256


def _make_deg(Npad, E):
  EperW = E // (_NC * _NS)
  NB = (EperW + 15) // 16
  mesh = plsc.VectorSubcoreMesh(core_axis_name="c", subcore_axis_name="s")

  def body(dsth, out, dstv, hist):
    c = lax.axis_index("c")
    s = lax.axis_index("s")
    wid = s * _NC + c
    pltpu.sync_copy(dsth.at[pl.ds(wid * EperW, EperW)],
                    dstv.at[pl.ds(0, EperW)])
    zero = jnp.zeros((16,), jnp.float32)

    def zb(i, _):
      hist[pl.ds(i * 16, 16)] = zero
      return 0

    lax.fori_loop(0, Npad // 16, zb, 0)
    ones = jnp.ones((16,), jnp.float32)
    lanes = lax.iota(jnp.int32, 16)

    def sb(i, _):
      idx = dstv[pl.ds(i * 16, 16)]
      m = (i * 16 + lanes) < EperW
      idx = jnp.where(m, idx, 0)
      plsc.addupdate_scatter(hist, [idx], ones, mask=m)
      return 0

    lax.fori_loop(0, NB, sb, 0)
    pltpu.sync_copy(hist, out.at[wid])

  return pl.kernel(
      body,
      out_type=jax.ShapeDtypeStruct((_NC * _NS, Npad), jnp.float32),
      mesh=mesh,
      scratch_types=[
          pltpu.VMEM((NB * 16,), jnp.int32),
          pltpu.VMEM((Npad,), jnp.float32),
      ],
      compiler_params=pltpu.CompilerParams(needs_layout_passes=False),
  )


def _make_agg(Npad, E, W, ACCR, SUB, K):
  NT = _NC * _NS
  RT = Npad // NT
  NSEG = 16 if W <= 128 else 25
  EperS = E // NSEG
  CAP = EperS + 16
  REG = NSEG * CAP + CAP
  assert RT == ACCR * SUB and EperS % 16 == 0 and K % 16 == 0
  SENT = jnp.int32(RT << 14)
  mesh = plsc.VectorSubcoreMesh(core_axis_name="c", subcore_axis_name="s")

  def body(vals, srch, dsth, outf, creg, sbuf, dbuf, cbuf, sec, rsrc, rsrc2,
           rsrc3, rsrc4, gbuf, gbuf2, gbuf3, gbuf4, acc, sem, sem2, sem3,
           sem4):
    c = lax.axis_index("c")
    s = lax.axis_index("s")
    wid = s * _NC + c
    rt_lo = wid * RT
    lanes = lax.iota(jnp.int32, 16)

    def seg_body(g, off):
      pltpu.sync_copy(srch.at[pl.ds(g * EperS, EperS)], sbuf)
      pltpu.sync_copy(dsth.at[pl.ds(g * EperS, EperS)],
                      dbuf.at[pl.ds(0, EperS)])

      def cb(i, cnt):
        d = dbuf[pl.ds(i * 16, 16)] - rt_lo
        m = (d >= 0) & (d < RT)
        pk = sbuf[pl.ds(i * 16, 16)] | (d << 14)
        plsc.store_compressed(cbuf.at[pl.ds(cnt, 16)], pk, mask=m)
        return cnt + plsc.all_reduce_population_count(m)[0]

      cnt = lax.fori_loop(0, EperS // 16, cb, jnp.int32(0))
      cbuf[pl.ds(cnt, 16)] = jnp.full((16,), SENT, jnp.int32)
      pltpu.sync_copy(
          cbuf, creg.at[pl.ds(pl.multiple_of(wid * REG + off, 8), CAP)])
      return off + ((cnt + 7) & jnp.int32(-8))

    total = lax.fori_loop(0, NSEG, seg_body, jnp.int32(0))

    def sub_body(sub, _):
      base_d = sub * ACCR
      dummy = jnp.zeros((16,), jnp.int32) + ((base_d + ACCR) << 14)

      def zb(i, _):
        acc[pl.ds(i * 16, 16)] = jnp.zeros((16,), jnp.float32)
        return 0

      lax.fori_loop(0, ((ACCR + 1) * W) // 16, zb, 0)
      nb1 = (total + CAP - 1) // CAP

      def rb_body(t, _):
        pltpu.sync_copy(
            creg.at[pl.ds(pl.multiple_of(wid * REG + t * CAP, 8), CAP)], dbuf)
        hi = jnp.minimum(total - t * CAP, CAP)

        def sc(i, sc_cnt):
          pk = dbuf[pl.ds(i * 16, 16)]
          dl = pk >> 14
          m = ((i * 16 + lanes) < hi) & (dl >= base_d) & (dl < base_d + ACCR)
          plsc.store_compressed(sec.at[pl.ds(sc_cnt, 16)], pk, mask=m)
          return sc_cnt + plsc.all_reduce_population_count(m)[0]

        scnt = lax.fori_loop(0, (hi + 15) // 16, sc, jnp.int32(0))
        for j in range(K // 16):
          sec[pl.ds(scnt + j * 16, 16)] = dummy
        nb2 = (scnt + K - 1) // K

        def fire(i, rs, gb, sm):
          for j in range(K // 16):
            pv = sec[pl.ds(i * K + j * 16, 16)]
            rs[pl.ds(j * 16, 16)] = pv & 0x3FFF
          pltpu.async_copy(vals.at[rs], gb, sm)

        def wait_g(rs, gb, sm):
          pltpu.make_async_copy(vals.at[rs], gb, sm).wait()

        def accum(i, gb):
          bvecs = []
          for j in range(K // 16):
            pv = sec[pl.ds(i * K + j * 16, 16)]
            dv = ((pv >> 14) - base_d) * W
            for r in range(16):
              lane_r = jnp.full((16,), r, jnp.int32)
              bvecs.append(
                  lanes + dv.at[lane_r].get(mode="promise_in_bounds"))

          def jb(j, _):
            col = j * 16
            for r in range(K):
              plsc.addupdate_scatter(acc, [bvecs[r] + col],
                                     gb[r, pl.ds(col, 16)])
            return 0

          lax.fori_loop(0, W // 16, jb, 0)

        rss = (rsrc, rsrc2, rsrc3, rsrc4)
        gbs = (gbuf, gbuf2, gbuf3, gbuf4)
        sms = (sem, sem2, sem3, sem4)
        for t in range(3):
          @pl.when(t < nb2)
          def _(t=t):
            fire(t, rss[t], gbs[t], sms[t])

        def drain4(b4, _):
          b = 4 * b4
          for t in range(4):
            @pl.when(b + t < nb2)
            def _(t=t):
              wait_g(rss[t], gbs[t], sms[t])

              @pl.when(b + t + 3 < nb2)
              def _():
                u = (t + 3) % 4
                fire(b + t + 3, rss[u], gbs[u], sms[u])

              accum(b + t, gbs[t])
          return 0

        lax.fori_loop(0, (nb2 + 3) // 4, drain4, 0)
        return 0

      lax.fori_loop(0, nb1, rb_body, 0)
      pltpu.sync_copy(
          acc.at[pl.ds(0, ACCR * W)],
          outf.at[pl.ds(pl.multiple_of((rt_lo + base_d) * W, 8), ACCR * W)])
      return 0

    lax.fori_loop(0, SUB, sub_body, 0)

  return pl.kernel(
      body,
      out_type=(
          jax.ShapeDtypeStruct((Npad * W,), jnp.float32),
          jax.ShapeDtypeStruct((NT * REG,), jnp.int32),
      ),
      mesh=mesh,
      scratch_types=[
          pltpu.VMEM((EperS,), jnp.int32),
          pltpu.VMEM((CAP,), jnp.int32),
          pltpu.VMEM((CAP,), jnp.int32),
          pltpu.VMEM((CAP + K,), jnp.int32),
          pltpu.VMEM((K,), jnp.int32),
          pltpu.VMEM((K,), jnp.int32),
          pltpu.VMEM((K,), jnp.int32),
          pltpu.VMEM((K,), jnp.int32),
          pltpu.VMEM((K, W), jnp.float32),
          pltpu.VMEM((K, W), jnp.float32),
          pltpu.VMEM((K, W), jnp.float32),
          pltpu.VMEM((K, W), jnp.float32),
          pltpu.VMEM(((ACCR + 1) * W,), jnp.float32),
          pltpu.SemaphoreType.DMA,
          pltpu.SemaphoreType.DMA,
          pltpu.SemaphoreType.DMA,
          pltpu.SemaphoreType.DMA,
      ],
      compiler_params=pltpu.CompilerParams(needs_layout_passes=False),
  )


def _prep(deg3, x_pad):
  Npad, IN = x_pad.shape
  NBLK = Npad // _BM

  def body(deg_ref, x_ref, dinv_ref, xs_ref):
    deg = jnp.sum(deg_ref[...], axis=0) + 1.0
    dinv = lax.rsqrt(deg)
    dinv_ref[...] = dinv
    xs_ref[...] = x_ref[...] * dinv

  return pl.pallas_call(
      body,
      grid=(NBLK,),
      in_specs=[
          pl.BlockSpec((_NC * _NS, _BM, 1), lambda i: (0, i, 0)),
          pl.BlockSpec((_BM, IN), lambda i: (i, 0)),
      ],
      out_specs=[
          pl.BlockSpec((_BM, 1), lambda i: (i, 0)),
          pl.BlockSpec((_BM, IN), lambda i: (i, 0)),
      ],
      out_shape=[
          jax.ShapeDtypeStruct((Npad, 1), jnp.float32),
          jax.ShapeDtypeStruct((Npad, IN), jnp.float32),
      ],
  )(deg3, x_pad)


def _mm1(scat1, xs, dinv, W1, b1):
  Npad, IN = xs.shape
  H = W1.shape[1]
  NBLK = Npad // _BM

  def body(s_ref, x_ref, d_ref, w_ref, b_ref, o_ref):
    dv = d_ref[...]
    a = (s_ref[...] + x_ref[...]) * dv
    z = jnp.dot(a, w_ref[...], preferred_element_type=jnp.float32) + b_ref[...]
    o_ref[...] = jnp.maximum(z, 0.0) * dv

  return pl.pallas_call(
      body,
      grid=(NBLK,),
      in_specs=[
          pl.BlockSpec((_BM, IN), lambda i: (i, 0)),
          pl.BlockSpec((_BM, IN), lambda i: (i, 0)),
          pl.BlockSpec((_BM, 1), lambda i: (i, 0)),
          pl.BlockSpec((IN, H), lambda i: (0, 0)),
          pl.BlockSpec((1, H), lambda i: (0, 0)),
      ],
      out_specs=pl.BlockSpec((_BM, H), lambda i: (i, 0)),
      out_shape=jax.ShapeDtypeStruct((Npad, H), jnp.float32),
  )(scat1, xs, dinv, W1, b1.reshape(1, -1))


def _mm2(scat2, h1s, dinv, W2, b2, Wc):
  Npad, H = h1s.shape
  OUT = Wc.shape[1]
  NBLK = Npad // _BM

  def body(s_ref, h_ref, d_ref, w_ref, b_ref, wc_ref, o_ref):
    dv = d_ref[...]
    a = (s_ref[...] + h_ref[...]) * dv
    z = jnp.dot(a, w_ref[...], preferred_element_type=jnp.float32) + b_ref[...]
    h2 = jnp.maximum(z, 0.0)
    o_ref[...] = jnp.dot(h2, wc_ref[...], preferred_element_type=jnp.float32) * dv

  return pl.pallas_call(
      body,
      grid=(NBLK,),
      in_specs=[
          pl.BlockSpec((_BM, H), lambda i: (i, 0)),
          pl.BlockSpec((_BM, H), lambda i: (i, 0)),
          pl.BlockSpec((_BM, 1), lambda i: (i, 0)),
          pl.BlockSpec((H, H), lambda i: (0, 0)),
          pl.BlockSpec((1, H), lambda i: (0, 0)),
          pl.BlockSpec((H, OUT), lambda i: (0, 0)),
      ],
      out_specs=pl.BlockSpec((_BM, OUT), lambda i: (i, 0)),
      out_shape=jax.ShapeDtypeStruct((Npad, OUT), jnp.float32),
  )(scat2, h1s, dinv, W2, b2.reshape(1, -1), Wc)


def _wc(W3, Wl):
  def body(w3_ref, wl_ref, o_ref):
    o_ref[...] = jnp.dot(w3_ref[...], wl_ref[...],
                         preferred_element_type=jnp.float32)

  return pl.pallas_call(
      body,
      out_shape=jax.ShapeDtypeStruct((W3.shape[0], Wl.shape[1]), jnp.float32),
  )(W3, Wl)


def _pool(scat3, ms, dinv, batch3, b3, Wl, bl, G):
  Npad, OUT = scat3.shape
  H = Wl.shape[0]
  NBLK = Npad // _BM

  def body(s_ref, m_ref, d_ref, b_ref, b3_ref, wl_ref, bl_ref, o_ref,
           sums, cnts):
    i = pl.program_id(0)

    @pl.when(i == 0)
    def _():
      sums[...] = jnp.zeros_like(sums)
      cnts[...] = jnp.zeros_like(cnts)

    a3 = (s_ref[...] + m_ref[...]) * d_ref[...]
    b = b_ref[0]
    oh = (lax.broadcasted_iota(jnp.int32, (G, _BM), 0) == b).astype(jnp.float32)
    sums[...] += jnp.dot(oh, a3, preferred_element_type=jnp.float32)
    cnts[...] += jnp.dot(oh, jnp.ones((_BM, OUT), jnp.float32),
                         preferred_element_type=jnp.float32)

    @pl.when(i == NBLK - 1)
    def _():
      c = cnts[...]
      bc = jnp.dot(b3_ref[...], wl_ref[...], preferred_element_type=jnp.float32)
      o_ref[...] = (sums[...] / jnp.maximum(c, 1.0)
                    + jnp.where(c > 0.0, bc, 0.0) + bl_ref[...])

  return pl.pallas_call(
      body,
      grid=(NBLK,),
      in_specs=[
          pl.BlockSpec((_BM, OUT), lambda i: (i, 0)),
          pl.BlockSpec((_BM, OUT), lambda i: (i, 0)),
          pl.BlockSpec((_BM, 1), lambda i: (i, 0)),
          pl.BlockSpec((1, 1, _BM), lambda i: (i, 0, 0)),
          pl.BlockSpec((1, H), lambda i: (0, 0)),
          pl.BlockSpec((H, OUT), lambda i: (0, 0)),
          pl.BlockSpec((1, OUT), lambda i: (0, 0)),
      ],
      out_specs=pl.BlockSpec((G, OUT), lambda i: (0, 0)),
      out_shape=jax.ShapeDtypeStruct((G, OUT), jnp.float32),
      scratch_shapes=[
          pltpu.VMEM((G, OUT), jnp.float32),
          pltpu.VMEM((G, OUT), jnp.float32),
      ],
  )(scat3, ms, dinv, batch3, b3.reshape(1, -1), Wl, bl.reshape(1, -1))


def kernel(x, edge_index, batch, W1, b1, W2, b2, W3, b3, Wl, bl):
  N, IN = x.shape
  E = edge_index.shape[1]
  H = W1.shape[1]
  OUT = Wl.shape[1]
  G = 64
  Npad = ((N + 2559) // 2560) * 2560

  src = edge_index[0]
  dst = edge_index[1]
  x_pad = jnp.pad(x.astype(jnp.float32), ((0, Npad - N), (0, 0)))
  batch_pad = jnp.pad(batch, (0, Npad - N), constant_values=G)
  batch3 = batch_pad.reshape(Npad // _BM, 1, _BM)

  deg = _make_deg(Npad, E)(dst)
  dinv, xs = _prep(deg.reshape(_NC * _NS, Npad, 1), x_pad)

  RT = Npad // (_NC * _NS)
  scat1, _ = _make_agg(Npad, E, IN, RT // 2, 2, 32)(xs, src, dst)
  h1s = _mm1(scat1.reshape(Npad, IN), xs, dinv, W1, b1)

  scat2, _ = _make_agg(Npad, E, H, RT // 10, 10, 16)(h1s, src, dst)
  Wc = _wc(W3, Wl)
  Wcp = jnp.pad(Wc, ((0, 0), (0, 128 - OUT)))
  ms128 = _mm2(scat2.reshape(Npad, H), h1s, dinv, W2, b2, Wcp)

  scat3, _ = _make_agg(Npad, E, 128, RT, 1, 64)(ms128, src, dst)
  return _pool(scat3.reshape(Npad, 128)[:, :OUT], ms128[:, :OUT], dinv,
               batch3, b3, Wl, bl, G)


# speedup vs baseline: 1.0695x; 1.0552x over previous
import functools

import jax
import jax.numpy as jnp
from jax import lax
from jax.experimental import pallas as pl
from jax.experimental.pallas import tpu as pltpu
from jax.experimental.pallas import tpu_sc as plsc

_NC = 2
_NS = 16
_BM = 256


def _make_deg(Npad, E):
  EperW = E // (_NC * _NS)
  NB = (EperW + 15) // 16
  mesh = plsc.VectorSubcoreMesh(core_axis_name="c", subcore_axis_name="s")

  def body(dsth, out, dstv, hist):
    c = lax.axis_index("c")
    s = lax.axis_index("s")
    wid = s * _NC + c
    pltpu.sync_copy(dsth.at[pl.ds(wid * EperW, EperW)],
                    dstv.at[pl.ds(0, EperW)])
    zero = jnp.zeros((16,), jnp.float32)

    def zb(i, _):
      hist[pl.ds(i * 16, 16)] = zero
      return 0

    lax.fori_loop(0, Npad // 16, zb, 0)
    ones = jnp.ones((16,), jnp.float32)
    lanes = lax.iota(jnp.int32, 16)

    def sb(i, _):
      idx = dstv[pl.ds(i * 16, 16)]
      m = (i * 16 + lanes) < EperW
      idx = jnp.where(m, idx, 0)
      plsc.addupdate_scatter(hist, [idx], ones, mask=m)
      return 0

    lax.fori_loop(0, NB, sb, 0)
    pltpu.sync_copy(hist, out.at[wid])

  return pl.kernel(
      body,
      out_type=jax.ShapeDtypeStruct((_NC * _NS, Npad), jnp.float32),
      mesh=mesh,
      scratch_types=[
          pltpu.VMEM((NB * 16,), jnp.int32),
          pltpu.VMEM((Npad,), jnp.float32),
      ],
      compiler_params=pltpu.CompilerParams(needs_layout_passes=False),
  )


def _make_agg(Npad, E, W, ACCR, SUB, K, vdt=jnp.float32):
  NT = _NC * _NS
  RT = Npad // NT
  NSEG = 16 if W <= 128 else 25
  EperS = E // NSEG
  CAP = EperS + 16
  REG = NSEG * CAP + CAP
  assert RT == ACCR * SUB and EperS % 16 == 0 and K % 16 == 0
  GW = W if vdt == jnp.float32 else W // 2
  GDT = jnp.float32 if vdt == jnp.float32 else jnp.int32
  SENT = jnp.int32(RT << 14)
  mesh = plsc.VectorSubcoreMesh(core_axis_name="c", subcore_axis_name="s")

  def body(vals, srch, dsth, outf, creg, sbuf, dbuf, cbuf, sec, rsrc, rsrc2,
           rsrc3, rsrc4, gbuf, gbuf2, gbuf3, gbuf4, acc, sem, sem2, sem3,
           sem4):
    c = lax.axis_index("c")
    s = lax.axis_index("s")
    wid = s * _NC + c
    rt_lo = wid * RT
    lanes = lax.iota(jnp.int32, 16)

    def seg_body(g, off):
      pltpu.sync_copy(srch.at[pl.ds(g * EperS, EperS)], sbuf)
      pltpu.sync_copy(dsth.at[pl.ds(g * EperS, EperS)],
                      dbuf.at[pl.ds(0, EperS)])

      def cb(i, cnt):
        d = dbuf[pl.ds(i * 16, 16)] - rt_lo
        m = (d >= 0) & (d < RT)
        pk = sbuf[pl.ds(i * 16, 16)] | (d << 14)
        plsc.store_compressed(cbuf.at[pl.ds(cnt, 16)], pk, mask=m)
        return cnt + plsc.all_reduce_population_count(m)[0]

      cnt = lax.fori_loop(0, EperS // 16, cb, jnp.int32(0))
      cbuf[pl.ds(cnt, 16)] = jnp.full((16,), SENT, jnp.int32)
      pltpu.sync_copy(
          cbuf, creg.at[pl.ds(pl.multiple_of(wid * REG + off, 8), CAP)])
      return off + ((cnt + 7) & jnp.int32(-8))

    total = lax.fori_loop(0, NSEG, seg_body, jnp.int32(0))

    def sub_body(sub, _):
      base_d = sub * ACCR
      dummy = jnp.zeros((16,), jnp.int32) + ((base_d + ACCR) << 14)

      def zb(i, _):
        acc[pl.ds(i * 16, 16)] = jnp.zeros((16,), jnp.float32)
        return 0

      lax.fori_loop(0, ((ACCR + 1) * W) // 16, zb, 0)
      nb1 = (total + CAP - 1) // CAP

      def rb_body(t, _):
        pltpu.sync_copy(
            creg.at[pl.ds(pl.multiple_of(wid * REG + t * CAP, 8), CAP)], dbuf)
        hi = jnp.minimum(total - t * CAP, CAP)

        def sc(i, sc_cnt):
          pk = dbuf[pl.ds(i * 16, 16)]
          dl = pk >> 14
          m = ((i * 16 + lanes) < hi) & (dl >= base_d) & (dl < base_d + ACCR)
          plsc.store_compressed(sec.at[pl.ds(sc_cnt, 16)], pk, mask=m)
          return sc_cnt + plsc.all_reduce_population_count(m)[0]

        scnt = lax.fori_loop(0, (hi + 15) // 16, sc, jnp.int32(0))
        for j in range(K // 16):
          sec[pl.ds(scnt + j * 16, 16)] = dummy
        nb2 = (scnt + K - 1) // K

        def fire(i, rs, gb, sm):
          for j in range(K // 16):
            pv = sec[pl.ds(i * K + j * 16, 16)]
            rs[pl.ds(j * 16, 16)] = pv & 0x3FFF
          pltpu.async_copy(vals.at[rs], gb, sm)

        def wait_g(rs, gb, sm):
          pltpu.make_async_copy(vals.at[rs], gb, sm).wait()

        def accum(i, gb):
          bvecs = []
          lstride = lanes if vdt == jnp.float32 else lanes * 2
          for j in range(K // 16):
            pv = sec[pl.ds(i * K + j * 16, 16)]
            dv = ((pv >> 14) - base_d) * W
            for r in range(16):
              lane_r = jnp.full((16,), r, jnp.int32)
              bvecs.append(
                  lstride + dv.at[lane_r].get(mode="promise_in_bounds"))

          if vdt == jnp.float32:
            def jb(j, _):
              col = j * 16
              for r in range(K):
                plsc.addupdate_scatter(acc, [bvecs[r] + col],
                                       gb[r, pl.ds(col, 16)])
              return 0

            lax.fori_loop(0, W // 16, jb, 0)
          else:
            def jb(j, _):
              col = j * 32
              for r in range(K):
                x32 = plsc.bitcast(gb[r, pl.ds(j * 16, 16)], jnp.bfloat16)
                a, b = plsc.unpack(x32, format=plsc.PackFormat.INTERLEAVED)
                plsc.addupdate_scatter(acc, [bvecs[r] + col], a)
                plsc.addupdate_scatter(acc, [bvecs[r] + (col + 1)], b)
              return 0

            lax.fori_loop(0, W // 32, jb, 0)

        rss = (rsrc, rsrc2, rsrc3, rsrc4)
        gbs = (gbuf, gbuf2, gbuf3, gbuf4)
        sms = (sem, sem2, sem3, sem4)
        for t in range(3):
          @pl.when(t < nb2)
          def _(t=t):
            fire(t, rss[t], gbs[t], sms[t])

        def drain4(b4, _):
          b = 4 * b4
          for t in range(4):
            @pl.when(b + t < nb2)
            def _(t=t):
              wait_g(rss[t], gbs[t], sms[t])

              @pl.when(b + t + 3 < nb2)
              def _():
                u = (t + 3) % 4
                fire(b + t + 3, rss[u], gbs[u], sms[u])

              accum(b + t, gbs[t])
          return 0

        lax.fori_loop(0, (nb2 + 3) // 4, drain4, 0)
        return 0

      lax.fori_loop(0, nb1, rb_body, 0)
      pltpu.sync_copy(
          acc.at[pl.ds(0, ACCR * W)],
          outf.at[pl.ds(pl.multiple_of((rt_lo + base_d) * W, 8), ACCR * W)])
      return 0

    lax.fori_loop(0, SUB, sub_body, 0)

  return pl.kernel(
      body,
      out_type=(
          jax.ShapeDtypeStruct((Npad * W,), jnp.float32),
          jax.ShapeDtypeStruct((NT * REG,), jnp.int32),
      ),
      mesh=mesh,
      scratch_types=[
          pltpu.VMEM((EperS,), jnp.int32),
          pltpu.VMEM((CAP,), jnp.int32),
          pltpu.VMEM((CAP,), jnp.int32),
          pltpu.VMEM((CAP + K,), jnp.int32),
          pltpu.VMEM((K,), jnp.int32),
          pltpu.VMEM((K,), jnp.int32),
          pltpu.VMEM((K,), jnp.int32),
          pltpu.VMEM((K,), jnp.int32),
          pltpu.VMEM((K, GW), GDT),
          pltpu.VMEM((K, GW), GDT),
          pltpu.VMEM((K, GW), GDT),
          pltpu.VMEM((K, GW), GDT),
          pltpu.VMEM(((ACCR + 1) * W,), jnp.float32),
          pltpu.SemaphoreType.DMA,
          pltpu.SemaphoreType.DMA,
          pltpu.SemaphoreType.DMA,
          pltpu.SemaphoreType.DMA,
      ],
      compiler_params=pltpu.CompilerParams(needs_layout_passes=False),
  )


def _prep(deg3, x_pad):
  Npad, IN = x_pad.shape
  NBLK = Npad // _BM

  def body(deg_ref, x_ref, dinv_ref, xs_ref):
    deg = jnp.sum(deg_ref[...], axis=0) + 1.0
    dinv = lax.rsqrt(deg)
    dinv_ref[...] = dinv
    xs_ref[...] = x_ref[...] * dinv

  return pl.pallas_call(
      body,
      grid=(NBLK,),
      in_specs=[
          pl.BlockSpec((_NC * _NS, _BM, 1), lambda i: (0, i, 0)),
          pl.BlockSpec((_BM, IN), lambda i: (i, 0)),
      ],
      out_specs=[
          pl.BlockSpec((_BM, 1), lambda i: (i, 0)),
          pl.BlockSpec((_BM, IN), lambda i: (i, 0)),
      ],
      out_shape=[
          jax.ShapeDtypeStruct((Npad, 1), jnp.float32),
          jax.ShapeDtypeStruct((Npad, IN), jnp.float32),
      ],
  )(deg3, x_pad)


def _mm1(scat1, xs, dinv, W1, b1):
  Npad, IN = xs.shape
  H = W1.shape[1]
  NBLK = Npad // _BM

  def body(s_ref, x_ref, d_ref, w_ref, b_ref, o_ref, ob_ref):
    dv = d_ref[...]
    a = (s_ref[...] + x_ref[...]) * dv
    z = jnp.dot(a, w_ref[...], preferred_element_type=jnp.float32) + b_ref[...]
    h = jnp.maximum(z, 0.0) * dv
    o_ref[...] = h
    ob_ref[...] = h.astype(jnp.bfloat16)

  return pl.pallas_call(
      body,
      grid=(NBLK,),
      in_specs=[
          pl.BlockSpec((_BM, IN), lambda i: (i, 0)),
          pl.BlockSpec((_BM, IN), lambda i: (i, 0)),
          pl.BlockSpec((_BM, 1), lambda i: (i, 0)),
          pl.BlockSpec((IN, H), lambda i: (0, 0)),
          pl.BlockSpec((1, H), lambda i: (0, 0)),
      ],
      out_specs=[
          pl.BlockSpec((_BM, H), lambda i: (i, 0)),
          pl.BlockSpec((_BM, H), lambda i: (i, 0)),
      ],
      out_shape=[
          jax.ShapeDtypeStruct((Npad, H), jnp.float32),
          jax.ShapeDtypeStruct((Npad, H), jnp.bfloat16),
      ],
  )(scat1, xs, dinv, W1, b1.reshape(1, -1))


def _mm2(scat2, h1s, dinv, W2, b2, Wc):
  Npad, H = h1s.shape
  OUT = Wc.shape[1]
  NBLK = Npad // _BM

  def body(s_ref, h_ref, d_ref, w_ref, b_ref, wc_ref, o_ref, ob_ref):
    dv = d_ref[...]
    a = (s_ref[...] + h_ref[...]) * dv
    z = jnp.dot(a, w_ref[...], preferred_element_type=jnp.float32) + b_ref[...]
    h2 = jnp.maximum(z, 0.0)
    m = jnp.dot(h2, wc_ref[...], preferred_element_type=jnp.float32) * dv
    o_ref[...] = m
    ob_ref[...] = m.astype(jnp.bfloat16)

  return pl.pallas_call(
      body,
      grid=(NBLK,),
      in_specs=[
          pl.BlockSpec((_BM, H), lambda i: (i, 0)),
          pl.BlockSpec((_BM, H), lambda i: (i, 0)),
          pl.BlockSpec((_BM, 1), lambda i: (i, 0)),
          pl.BlockSpec((H, H), lambda i: (0, 0)),
          pl.BlockSpec((1, H), lambda i: (0, 0)),
          pl.BlockSpec((H, OUT), lambda i: (0, 0)),
      ],
      out_specs=[
          pl.BlockSpec((_BM, OUT), lambda i: (i, 0)),
          pl.BlockSpec((_BM, OUT), lambda i: (i, 0)),
      ],
      out_shape=[
          jax.ShapeDtypeStruct((Npad, OUT), jnp.float32),
          jax.ShapeDtypeStruct((Npad, OUT), jnp.bfloat16),
      ],
  )(scat2, h1s, dinv, W2, b2.reshape(1, -1), Wc)


def _wc(W3, Wl):
  def body(w3_ref, wl_ref, o_ref):
    o_ref[...] = jnp.dot(w3_ref[...], wl_ref[...],
                         preferred_element_type=jnp.float32)

  return pl.pallas_call(
      body,
      out_shape=jax.ShapeDtypeStruct((W3.shape[0], Wl.shape[1]), jnp.float32),
  )(W3, Wl)


def _pool(scat3, ms, dinv, batch3, b3, Wl, bl, G):
  Npad, OUT = scat3.shape
  H = Wl.shape[0]
  NBLK = Npad // _BM

  def body(s_ref, m_ref, d_ref, b_ref, b3_ref, wl_ref, bl_ref, o_ref,
           sums, cnts):
    i = pl.program_id(0)

    @pl.when(i == 0)
    def _():
      sums[...] = jnp.zeros_like(sums)
      cnts[...] = jnp.zeros_like(cnts)

    a3 = (s_ref[...] + m_ref[...]) * d_ref[...]
    b = b_ref[0]
    oh = (lax.broadcasted_iota(jnp.int32, (G, _BM), 0) == b).astype(jnp.float32)
    sums[...] += jnp.dot(oh, a3, preferred_element_type=jnp.float32)
    cnts[...] += jnp.dot(oh, jnp.ones((_BM, OUT), jnp.float32),
                         preferred_element_type=jnp.float32)

    @pl.when(i == NBLK - 1)
    def _():
      c = cnts[...]
      bc = jnp.dot(b3_ref[...], wl_ref[...], preferred_element_type=jnp.float32)
      o_ref[...] = (sums[...] / jnp.maximum(c, 1.0)
                    + jnp.where(c > 0.0, bc, 0.0) + bl_ref[...])

  return pl.pallas_call(
      body,
      grid=(NBLK,),
      in_specs=[
          pl.BlockSpec((_BM, OUT), lambda i: (i, 0)),
          pl.BlockSpec((_BM, OUT), lambda i: (i, 0)),
          pl.BlockSpec((_BM, 1), lambda i: (i, 0)),
          pl.BlockSpec((1, 1, _BM), lambda i: (i, 0, 0)),
          pl.BlockSpec((1, H), lambda i: (0, 0)),
          pl.BlockSpec((H, OUT), lambda i: (0, 0)),
          pl.BlockSpec((1, OUT), lambda i: (0, 0)),
      ],
      out_specs=pl.BlockSpec((G, OUT), lambda i: (0, 0)),
      out_shape=jax.ShapeDtypeStruct((G, OUT), jnp.float32),
      scratch_shapes=[
          pltpu.VMEM((G, OUT), jnp.float32),
          pltpu.VMEM((G, OUT), jnp.float32),
      ],
  )(scat3, ms, dinv, batch3, b3.reshape(1, -1), Wl, bl.reshape(1, -1))


def kernel(x, edge_index, batch, W1, b1, W2, b2, W3, b3, Wl, bl):
  N, IN = x.shape
  E = edge_index.shape[1]
  H = W1.shape[1]
  OUT = Wl.shape[1]
  G = 64
  Npad = ((N + 2559) // 2560) * 2560

  src = edge_index[0]
  dst = edge_index[1]
  x_pad = jnp.pad(x.astype(jnp.float32), ((0, Npad - N), (0, 0)))
  batch_pad = jnp.pad(batch, (0, Npad - N), constant_values=G)
  batch3 = batch_pad.reshape(Npad // _BM, 1, _BM)

  deg = _make_deg(Npad, E)(dst)
  dinv, xs = _prep(deg.reshape(_NC * _NS, Npad, 1), x_pad)

  RT = Npad // (_NC * _NS)
  scat1, _ = _make_agg(Npad, E, IN, RT // 2, 2, 32)(xs, src, dst)
  h1s, h1b = _mm1(scat1.reshape(Npad, IN), xs, dinv, W1, b1)

  h1p = jax.lax.bitcast_convert_type(
      h1b.reshape(Npad, H // 2, 2), jnp.int32)
  scat2, _ = _make_agg(Npad, E, H, RT // 10, 10, 16, jnp.bfloat16)(
      h1p, src, dst)
  Wc = _wc(W3, Wl)
  Wcp = jnp.pad(Wc, ((0, 0), (0, 128 - OUT)))
  ms128, _msb = _mm2(scat2.reshape(Npad, H), h1s, dinv, W2, b2, Wcp)

  scat3, _ = _make_agg(Npad, E, 128, RT, 1, 64)(ms128, src, dst)
  return _pool(scat3.reshape(Npad, 128)[:, :OUT], ms128[:, :OUT], dinv,
               batch3, b3, Wl, bl, G)


# speedup vs baseline: 1.6467x; 1.5397x over previous
import functools

import jax
import jax.numpy as jnp
from jax import lax
from jax.experimental import pallas as pl
from jax.experimental.pallas import tpu as pltpu
from jax.experimental.pallas import tpu_sc as plsc

_NC = 2
_NS = 16
_BM = 256


def _make_deg(Npad, E):
  EperW = E // (_NC * _NS)
  NB = (EperW + 15) // 16
  mesh = plsc.VectorSubcoreMesh(core_axis_name="c", subcore_axis_name="s")

  def body(dsth, out, dstv, hist):
    c = lax.axis_index("c")
    s = lax.axis_index("s")
    wid = s * _NC + c
    pltpu.sync_copy(dsth.at[pl.ds(wid * EperW, EperW)],
                    dstv.at[pl.ds(0, EperW)])
    zero = jnp.zeros((16,), jnp.float32)

    def zb(i, _):
      hist[pl.ds(i * 16, 16)] = zero
      return 0

    lax.fori_loop(0, Npad // 16, zb, 0)
    ones = jnp.ones((16,), jnp.float32)
    lanes = lax.iota(jnp.int32, 16)

    def sb(i, _):
      idx = dstv[pl.ds(i * 16, 16)]
      m = (i * 16 + lanes) < EperW
      idx = jnp.where(m, idx, 0)
      plsc.addupdate_scatter(hist, [idx], ones, mask=m)
      return 0

    lax.fori_loop(0, NB, sb, 0)
    pltpu.sync_copy(hist, out.at[wid])

  return pl.kernel(
      body,
      out_type=jax.ShapeDtypeStruct((_NC * _NS, Npad), jnp.float32),
      mesh=mesh,
      scratch_types=[
          pltpu.VMEM((NB * 16,), jnp.int32),
          pltpu.VMEM((Npad,), jnp.float32),
      ],
      compiler_params=pltpu.CompilerParams(needs_layout_passes=False),
  )


def _make_agg(Npad, E, W, ACCR, SUB, K, vdt=jnp.float32):
  NT = _NC * _NS
  RT = Npad // NT
  NSEG = 16 if W <= 128 else 25
  EperS = E // NSEG
  CAP = EperS + 16
  REG = NSEG * CAP + CAP
  assert RT == ACCR * SUB and EperS % 16 == 0 and K % 16 == 0
  GW = W if vdt == jnp.float32 else W // 2
  GDT = jnp.float32 if vdt == jnp.float32 else jnp.int32
  SENT = jnp.int32(RT << 14)
  mesh = plsc.VectorSubcoreMesh(core_axis_name="c", subcore_axis_name="s")

  def body(vals, srch, dsth, outf, creg, sbuf, dbuf, cbuf, sec, rsrc, rsrc2,
           rsrc3, rsrc4, gbuf, gbuf2, gbuf3, gbuf4, acc, sem, sem2, sem3,
           sem4):
    c = lax.axis_index("c")
    s = lax.axis_index("s")
    wid = s * _NC + c
    rt_lo = wid * RT
    lanes = lax.iota(jnp.int32, 16)

    def seg_body(g, off):
      pltpu.sync_copy(srch.at[pl.ds(g * EperS, EperS)], sbuf)
      pltpu.sync_copy(dsth.at[pl.ds(g * EperS, EperS)],
                      dbuf.at[pl.ds(0, EperS)])

      def cb(i, cnt):
        d = dbuf[pl.ds(i * 16, 16)] - rt_lo
        m = (d >= 0) & (d < RT)
        pk = sbuf[pl.ds(i * 16, 16)] | (d << 14)
        plsc.store_compressed(cbuf.at[pl.ds(cnt, 16)], pk, mask=m)
        return cnt + plsc.all_reduce_population_count(m)[0]

      cnt = lax.fori_loop(0, EperS // 16, cb, jnp.int32(0))
      cbuf[pl.ds(cnt, 16)] = jnp.full((16,), SENT, jnp.int32)
      pltpu.sync_copy(
          cbuf, creg.at[pl.ds(pl.multiple_of(wid * REG + off, 8), CAP)])
      return off + ((cnt + 7) & jnp.int32(-8))

    total = lax.fori_loop(0, NSEG, seg_body, jnp.int32(0))

    def sub_body(sub, _):
      base_d = sub * ACCR
      dummy = jnp.zeros((16,), jnp.int32) + ((base_d + ACCR) << 14)

      def zb(i, _):
        acc[pl.ds(i * 16, 16)] = jnp.zeros((16,), jnp.float32)
        return 0

      lax.fori_loop(0, ((ACCR + 1) * W) // 16, zb, 0)
      nb1 = (total + CAP - 1) // CAP

      def rb_body(t, _):
        pltpu.sync_copy(
            creg.at[pl.ds(pl.multiple_of(wid * REG + t * CAP, 8), CAP)], dbuf)
        hi = jnp.minimum(total - t * CAP, CAP)

        def sc(i, sc_cnt):
          pk = dbuf[pl.ds(i * 16, 16)]
          dl = pk >> 14
          m = ((i * 16 + lanes) < hi) & (dl >= base_d) & (dl < base_d + ACCR)
          plsc.store_compressed(sec.at[pl.ds(sc_cnt, 16)], pk, mask=m)
          return sc_cnt + plsc.all_reduce_population_count(m)[0]

        scnt = lax.fori_loop(0, (hi + 15) // 16, sc, jnp.int32(0))
        for j in range(K // 16):
          sec[pl.ds(scnt + j * 16, 16)] = dummy
        nb2 = (scnt + K - 1) // K

        def fire(i, rs, gb, sm):
          for j in range(K // 16):
            pv = sec[pl.ds(i * K + j * 16, 16)]
            rs[pl.ds(j * 16, 16)] = pv & 0x3FFF
          pltpu.async_copy(vals.at[rs], gb, sm)

        def wait_g(rs, gb, sm):
          pltpu.make_async_copy(vals.at[rs], gb, sm).wait()

        def accum(i, gb):
          bvecs = []
          lstride = lanes if vdt == jnp.float32 else lanes * 2
          for j in range(K // 16):
            pv = sec[pl.ds(i * K + j * 16, 16)]
            dv = ((pv >> 14) - base_d) * W
            for r in range(16):
              lane_r = jnp.full((16,), r, jnp.int32)
              bvecs.append(
                  lstride + dv.at[lane_r].get(mode="promise_in_bounds"))

          if vdt == jnp.float32:
            @functools.partial(plsc.parallel_loop, 0, W // 16, unroll=2)
            def _jb(j):
              col = j * 16
              for r in range(K):
                plsc.addupdate_scatter(acc, [bvecs[r] + col],
                                       gb[r, pl.ds(col, 16)])
          else:
            @functools.partial(plsc.parallel_loop, 0, W // 32, unroll=2)
            def _jb(j):
              col = j * 32
              for r in range(K):
                x32 = plsc.bitcast(gb[r, pl.ds(j * 16, 16)], jnp.bfloat16)
                a, b = plsc.unpack(x32, format=plsc.PackFormat.INTERLEAVED)
                plsc.addupdate_scatter(acc, [bvecs[r] + col], a)
                plsc.addupdate_scatter(acc, [bvecs[r] + (col + 1)], b)

        rss = (rsrc, rsrc2, rsrc3, rsrc4)
        gbs = (gbuf, gbuf2, gbuf3, gbuf4)
        sms = (sem, sem2, sem3, sem4)
        for t in range(3):
          @pl.when(t < nb2)
          def _(t=t):
            fire(t, rss[t], gbs[t], sms[t])

        def drain4(b4, _):
          b = 4 * b4
          for t in range(4):
            @pl.when(b + t < nb2)
            def _(t=t):
              wait_g(rss[t], gbs[t], sms[t])

              @pl.when(b + t + 3 < nb2)
              def _():
                u = (t + 3) % 4
                fire(b + t + 3, rss[u], gbs[u], sms[u])

              accum(b + t, gbs[t])
          return 0

        lax.fori_loop(0, (nb2 + 3) // 4, drain4, 0)
        return 0

      lax.fori_loop(0, nb1, rb_body, 0)
      pltpu.sync_copy(
          acc.at[pl.ds(0, ACCR * W)],
          outf.at[pl.ds(pl.multiple_of((rt_lo + base_d) * W, 8), ACCR * W)])
      return 0

    lax.fori_loop(0, SUB, sub_body, 0)

  return pl.kernel(
      body,
      out_type=(
          jax.ShapeDtypeStruct((Npad * W,), jnp.float32),
          jax.ShapeDtypeStruct((NT * REG,), jnp.int32),
      ),
      mesh=mesh,
      scratch_types=[
          pltpu.VMEM((EperS,), jnp.int32),
          pltpu.VMEM((CAP,), jnp.int32),
          pltpu.VMEM((CAP,), jnp.int32),
          pltpu.VMEM((CAP + K,), jnp.int32),
          pltpu.VMEM((K,), jnp.int32),
          pltpu.VMEM((K,), jnp.int32),
          pltpu.VMEM((K,), jnp.int32),
          pltpu.VMEM((K,), jnp.int32),
          pltpu.VMEM((K, GW), GDT),
          pltpu.VMEM((K, GW), GDT),
          pltpu.VMEM((K, GW), GDT),
          pltpu.VMEM((K, GW), GDT),
          pltpu.VMEM(((ACCR + 1) * W,), jnp.float32),
          pltpu.SemaphoreType.DMA,
          pltpu.SemaphoreType.DMA,
          pltpu.SemaphoreType.DMA,
          pltpu.SemaphoreType.DMA,
      ],
      compiler_params=pltpu.CompilerParams(needs_layout_passes=False),
  )


def _prep(deg3, x_pad):
  Npad, IN = x_pad.shape
  NBLK = Npad // _BM

  def body(deg_ref, x_ref, dinv_ref, xs_ref):
    deg = jnp.sum(deg_ref[...], axis=0) + 1.0
    dinv = lax.rsqrt(deg)
    dinv_ref[...] = dinv
    xs_ref[...] = x_ref[...] * dinv

  return pl.pallas_call(
      body,
      grid=(NBLK,),
      in_specs=[
          pl.BlockSpec((_NC * _NS, _BM, 1), lambda i: (0, i, 0)),
          pl.BlockSpec((_BM, IN), lambda i: (i, 0)),
      ],
      out_specs=[
          pl.BlockSpec((_BM, 1), lambda i: (i, 0)),
          pl.BlockSpec((_BM, IN), lambda i: (i, 0)),
      ],
      out_shape=[
          jax.ShapeDtypeStruct((Npad, 1), jnp.float32),
          jax.ShapeDtypeStruct((Npad, IN), jnp.float32),
      ],
  )(deg3, x_pad)


def _mm1(scat1, xs, dinv, W1, b1):
  Npad, IN = xs.shape
  H = W1.shape[1]
  NBLK = Npad // _BM

  def body(s_ref, x_ref, d_ref, w_ref, b_ref, o_ref, ob_ref):
    dv = d_ref[...]
    a = (s_ref[...] + x_ref[...]) * dv
    z = jnp.dot(a, w_ref[...], preferred_element_type=jnp.float32) + b_ref[...]
    h = jnp.maximum(z, 0.0) * dv
    o_ref[...] = h
    ob_ref[...] = h.astype(jnp.bfloat16)

  return pl.pallas_call(
      body,
      grid=(NBLK,),
      in_specs=[
          pl.BlockSpec((_BM, IN), lambda i: (i, 0)),
          pl.BlockSpec((_BM, IN), lambda i: (i, 0)),
          pl.BlockSpec((_BM, 1), lambda i: (i, 0)),
          pl.BlockSpec((IN, H), lambda i: (0, 0)),
          pl.BlockSpec((1, H), lambda i: (0, 0)),
      ],
      out_specs=[
          pl.BlockSpec((_BM, H), lambda i: (i, 0)),
          pl.BlockSpec((_BM, H), lambda i: (i, 0)),
      ],
      out_shape=[
          jax.ShapeDtypeStruct((Npad, H), jnp.float32),
          jax.ShapeDtypeStruct((Npad, H), jnp.bfloat16),
      ],
  )(scat1, xs, dinv, W1, b1.reshape(1, -1))


def _mm2(scat2, h1s, dinv, W2, b2, Wc):
  Npad, H = h1s.shape
  OUT = Wc.shape[1]
  NBLK = Npad // _BM

  def body(s_ref, h_ref, d_ref, w_ref, b_ref, wc_ref, o_ref, ob_ref):
    dv = d_ref[...]
    a = (s_ref[...] + h_ref[...]) * dv
    z = jnp.dot(a, w_ref[...], preferred_element_type=jnp.float32) + b_ref[...]
    h2 = jnp.maximum(z, 0.0)
    m = jnp.dot(h2, wc_ref[...], preferred_element_type=jnp.float32) * dv
    o_ref[...] = m
    ob_ref[...] = m.astype(jnp.bfloat16)

  return pl.pallas_call(
      body,
      grid=(NBLK,),
      in_specs=[
          pl.BlockSpec((_BM, H), lambda i: (i, 0)),
          pl.BlockSpec((_BM, H), lambda i: (i, 0)),
          pl.BlockSpec((_BM, 1), lambda i: (i, 0)),
          pl.BlockSpec((H, H), lambda i: (0, 0)),
          pl.BlockSpec((1, H), lambda i: (0, 0)),
          pl.BlockSpec((H, OUT), lambda i: (0, 0)),
      ],
      out_specs=[
          pl.BlockSpec((_BM, OUT), lambda i: (i, 0)),
          pl.BlockSpec((_BM, OUT), lambda i: (i, 0)),
      ],
      out_shape=[
          jax.ShapeDtypeStruct((Npad, OUT), jnp.float32),
          jax.ShapeDtypeStruct((Npad, OUT), jnp.bfloat16),
      ],
  )(scat2, h1s, dinv, W2, b2.reshape(1, -1), Wc)


def _wc(W3, Wl):
  def body(w3_ref, wl_ref, o_ref):
    o_ref[...] = jnp.dot(w3_ref[...], wl_ref[...],
                         preferred_element_type=jnp.float32)

  return pl.pallas_call(
      body,
      out_shape=jax.ShapeDtypeStruct((W3.shape[0], Wl.shape[1]), jnp.float32),
  )(W3, Wl)


def _pool(scat3, ms, dinv, batch3, b3, Wl, bl, G):
  Npad, OUT = scat3.shape
  H = Wl.shape[0]
  NBLK = Npad // _BM

  def body(s_ref, m_ref, d_ref, b_ref, b3_ref, wl_ref, bl_ref, o_ref,
           sums, cnts):
    i = pl.program_id(0)

    @pl.when(i == 0)
    def _():
      sums[...] = jnp.zeros_like(sums)
      cnts[...] = jnp.zeros_like(cnts)

    a3 = (s_ref[...] + m_ref[...]) * d_ref[...]
    b = b_ref[0]
    oh = (lax.broadcasted_iota(jnp.int32, (G, _BM), 0) == b).astype(jnp.float32)
    sums[...] += jnp.dot(oh, a3, preferred_element_type=jnp.float32)
    cnts[...] += jnp.dot(oh, jnp.ones((_BM, OUT), jnp.float32),
                         preferred_element_type=jnp.float32)

    @pl.when(i == NBLK - 1)
    def _():
      c = cnts[...]
      bc = jnp.dot(b3_ref[...], wl_ref[...], preferred_element_type=jnp.float32)
      o_ref[...] = (sums[...] / jnp.maximum(c, 1.0)
                    + jnp.where(c > 0.0, bc, 0.0) + bl_ref[...])

  return pl.pallas_call(
      body,
      grid=(NBLK,),
      in_specs=[
          pl.BlockSpec((_BM, OUT), lambda i: (i, 0)),
          pl.BlockSpec((_BM, OUT), lambda i: (i, 0)),
          pl.BlockSpec((_BM, 1), lambda i: (i, 0)),
          pl.BlockSpec((1, 1, _BM), lambda i: (i, 0, 0)),
          pl.BlockSpec((1, H), lambda i: (0, 0)),
          pl.BlockSpec((H, OUT), lambda i: (0, 0)),
          pl.BlockSpec((1, OUT), lambda i: (0, 0)),
      ],
      out_specs=pl.BlockSpec((G, OUT), lambda i: (0, 0)),
      out_shape=jax.ShapeDtypeStruct((G, OUT), jnp.float32),
      scratch_shapes=[
          pltpu.VMEM((G, OUT), jnp.float32),
          pltpu.VMEM((G, OUT), jnp.float32),
      ],
  )(scat3, ms, dinv, batch3, b3.reshape(1, -1), Wl, bl.reshape(1, -1))


def kernel(x, edge_index, batch, W1, b1, W2, b2, W3, b3, Wl, bl):
  N, IN = x.shape
  E = edge_index.shape[1]
  H = W1.shape[1]
  OUT = Wl.shape[1]
  G = 64
  Npad = ((N + 2559) // 2560) * 2560

  src = edge_index[0]
  dst = edge_index[1]
  x_pad = jnp.pad(x.astype(jnp.float32), ((0, Npad - N), (0, 0)))
  batch_pad = jnp.pad(batch, (0, Npad - N), constant_values=G)
  batch3 = batch_pad.reshape(Npad // _BM, 1, _BM)

  deg = _make_deg(Npad, E)(dst)
  dinv, xs = _prep(deg.reshape(_NC * _NS, Npad, 1), x_pad)

  RT = Npad // (_NC * _NS)
  scat1, _ = _make_agg(Npad, E, IN, RT // 2, 2, 32)(xs, src, dst)
  h1s, h1b = _mm1(scat1.reshape(Npad, IN), xs, dinv, W1, b1)

  h1p = jax.lax.bitcast_convert_type(
      h1b.reshape(Npad, H // 2, 2), jnp.int32)
  scat2, _ = _make_agg(Npad, E, H, RT // 10, 10, 16, jnp.bfloat16)(
      h1p, src, dst)
  Wc = _wc(W3, Wl)
  Wcp = jnp.pad(Wc, ((0, 0), (0, 128 - OUT)))
  ms128, _msb = _mm2(scat2.reshape(Npad, H), h1s, dinv, W2, b2, Wcp)

  scat3, _ = _make_agg(Npad, E, 128, RT, 1, 64)(ms128, src, dst)
  return _pool(scat3.reshape(Npad, 128)[:, :OUT], ms128[:, :OUT], dinv,
               batch3, b3, Wl, bl, G)
